# baseline (speedup 1.0000x reference)
import jax
import jax.numpy as jnp
import numpy as np
from jax import lax
from jax.experimental import pallas as pl
from jax.experimental.pallas import tpu as pltpu
from jax.experimental.pallas import tpu_sc as plsc

T, N, E, D, C = 36, 10000, 320000, 128, 6
_PH = jax.lax.Precision.HIGHEST
NP = 10240
HD = D // 2
NC, NS = 2, 16
NW = NC * NS
EW = E // NW
CH = 128
NCH = 80
EWP = NCH * CH
STRIPE = NP // NS
W2 = 16

_TS0, _NTS, _NI = 6, 4, 12
_CNT = _NTS * _NI
_ZROWS = 40
_OFFS = (0, 7, 8, 9, 10, 11, 12, 13)


def _segsum(xa, xb, srco, dstr, zrows, zrows8, ones, with_deg):
    mesh = plsc.VectorSubcoreMesh(core_axis_name="c", subcore_axis_name="s")

    def body(xa_r, xb_r, srco_r, dstr_r, zrows_r, zrows8_r, ones_r,
             *out_and_scratch):
        if with_deg:
            (aggp_r, degp_r, idxs_v, idxd_v, rows_v, ones_v, zbuf_v, zbuf8_v,
             sem_a, sem_b, agg_sh, deg_sh) = out_and_scratch
        else:
            (aggp_r, idxs_v, idxd_v, rows_v, ones_v, zbuf_v, zbuf8_v,
             sem_a, sem_b, agg_sh) = out_and_scratch
            degp_r = deg_sh = None
        c = lax.axis_index("c")
        s = lax.axis_index("s")
        wid = s * NC + c
        pltpu.sync_copy(ones_r, ones_v)
        pltpu.sync_copy(zrows_r, zbuf_v)
        pltpu.sync_copy(zrows8_r, zbuf8_v)

        def step_t(t, carry):
            pltpu.sync_copy(srco_r.at[t, wid], idxs_v)
            pltpu.sync_copy(dstr_r.at[t, wid], idxd_v)
            for h, tab in ((0, xa_r), (1, xb_r)):
                pltpu.sync_copy(zbuf_v, agg_sh.at[pl.ds(s * STRIPE, STRIPE)])
                if h == 0 and with_deg:
                    pltpu.sync_copy(zbuf8_v,
                                    deg_sh.at[pl.ds(s * STRIPE, STRIPE)])
                plsc.subcore_barrier()

                pltpu.async_copy(tab.at[idxs_v.at[0]], rows_v.at[0], sem_a)

                def wait_rows(sem):
                    pltpu.make_async_copy(tab.at[idxs_v.at[0]],
                                          rows_v.at[0], sem).wait()

                def scat(j, buf):
                    pltpu.sync_copy(rows_v.at[buf], agg_sh.at[idxd_v.at[j]],
                                    add=True)
                    if h == 0 and with_deg:
                        pltpu.sync_copy(ones_v, deg_sh.at[idxd_v.at[j]],
                                        add=True)

                def step_j(jj, carry2):
                    j0 = 2 * jj
                    pltpu.async_copy(tab.at[idxs_v.at[j0 + 1]],
                                     rows_v.at[1], sem_b)
                    wait_rows(sem_a)
                    scat(j0, 0)

                    @pl.when(jj < NCH // 2 - 1)
                    def _():
                        pltpu.async_copy(tab.at[idxs_v.at[j0 + 2]],
                                         rows_v.at[0], sem_a)

                    wait_rows(sem_b)
                    scat(j0 + 1, 1)
                    return carry2

                lax.fori_loop(0, NCH // 2, step_j, 0)
                plsc.subcore_barrier()
                pltpu.sync_copy(agg_sh.at[pl.ds(s * STRIPE, STRIPE)],
                                aggp_r.at[h, t, c, pl.ds(s * STRIPE, STRIPE)])
                if h == 0 and with_deg:
                    pltpu.sync_copy(deg_sh.at[pl.ds(s * STRIPE, STRIPE)],
                                    degp_r.at[t, c, pl.ds(s * STRIPE, STRIPE)])
                plsc.subcore_barrier()
            return carry

        lax.fori_loop(0, T, step_t, 0)

    out_type = [jax.ShapeDtypeStruct((2, T, NC, NP, HD), jnp.float32)]
    scratch = [
        pltpu.VMEM((NCH, CH), jnp.int32),
        pltpu.VMEM((NCH, CH), jnp.int32),
        pltpu.VMEM((2, CH, HD), jnp.float32),
        pltpu.VMEM((CH, 8), jnp.float32),
        pltpu.VMEM((STRIPE, HD), jnp.float32),
        pltpu.VMEM((STRIPE, 8), jnp.float32),
        pltpu.SemaphoreType.DMA,
        pltpu.SemaphoreType.DMA,
        pltpu.VMEM_SHARED((NP, HD), jnp.float32),
    ]
    if with_deg:
        out_type.append(jax.ShapeDtypeStruct((T, NC, NP, 8), jnp.float32))
        scratch.append(pltpu.VMEM_SHARED((NP, 8), jnp.float32))

    f = pl.kernel(
        body,
        out_type=tuple(out_type),
        mesh=mesh,
        scratch_types=scratch,
        compiler_params=pltpu.CompilerParams(use_tc_tiling_on_sc=False),
    )
    return f(xa, xb, srco, dstr, zrows, zrows8, ones)


_BN = 2048


def _dense1_body(a00, a01, a10, a11, d0, d1, x, wl1t, wr1t, b1,
                 ha_ref, hb_ref):
    agg = jnp.concatenate([a00[0, 0, 0] + a01[0, 0, 0],
                           a10[0, 0, 0] + a11[0, 0, 0]], axis=1)
    deg = d0[0, 0, :, 0:1] + d1[0, 0, :, 0:1]
    aggm = agg / jnp.maximum(deg, 1.0)
    h = (jnp.dot(aggm, wl1t[...], preferred_element_type=jnp.float32)
         + jnp.dot(x[0], wr1t[...], preferred_element_type=jnp.float32)
         + b1[...])
    h = jnp.maximum(h, 0.0)
    ha_ref[0] = h[:, :HD]
    hb_ref[0] = h[:, HD:]


def _dense1(aggp, degp, xp, wl1t, wr1t, b1r):
    nb = NP // _BN
    return pl.pallas_call(
        _dense1_body,
        grid=(T, nb),
        in_specs=[
            pl.BlockSpec((1, 1, 1, _BN, HD), lambda t, r: (0, t, 0, r, 0)),
            pl.BlockSpec((1, 1, 1, _BN, HD), lambda t, r: (0, t, 1, r, 0)),
            pl.BlockSpec((1, 1, 1, _BN, HD), lambda t, r: (1, t, 0, r, 0)),
            pl.BlockSpec((1, 1, 1, _BN, HD), lambda t, r: (1, t, 1, r, 0)),
            pl.BlockSpec((1, 1, _BN, 8), lambda t, r: (t, 0, r, 0)),
            pl.BlockSpec((1, 1, _BN, 8), lambda t, r: (t, 1, r, 0)),
            pl.BlockSpec((1, _BN, D), lambda t, r: (t, r, 0)),
            pl.BlockSpec((D, D), lambda t, r: (0, 0)),
            pl.BlockSpec((D, D), lambda t, r: (0, 0)),
            pl.BlockSpec((1, D), lambda t, r: (0, 0)),
        ],
        out_specs=[
            pl.BlockSpec((1, _BN, HD), lambda t, r: (t, r, 0)),
            pl.BlockSpec((1, _BN, HD), lambda t, r: (t, r, 0)),
        ],
        out_shape=[
            jax.ShapeDtypeStruct((T, NP, HD), jnp.float32),
            jax.ShapeDtypeStruct((T, NP, HD), jnp.float32),
        ],
    )(aggp, aggp, aggp, aggp, degp, degp, xp, wl1t, wr1t, b1r)


def _pool_body(a00, a01, a10, a11, d0, d1, ha, hb, wl2a, wl2b, wr2a, wr2b,
               b2, z_ref):
    r = pl.program_id(1)
    deg = jnp.maximum(d0[0, 0, :, 0:1] + d1[0, 0, :, 0:1], 1.0)
    agg_a = (a00[0, 0, 0] + a01[0, 0, 0]) / deg
    agg_b = (a10[0, 0, 0] + a11[0, 0, 0]) / deg
    out2 = (jnp.dot(agg_a, wl2a[...], preferred_element_type=jnp.float32)
            + jnp.dot(agg_b, wl2b[...], preferred_element_type=jnp.float32)
            + jnp.dot(ha[0], wr2a[...], preferred_element_type=jnp.float32)
            + jnp.dot(hb[0], wr2b[...], preferred_element_type=jnp.float32)
            + b2[...])
    rows = lax.broadcasted_iota(jnp.int32, (_BN, 8), 0) + r * _BN
    out2 = jnp.where(rows < N, out2, -jnp.inf)
    zv = jnp.max(out2, axis=0, keepdims=True)
    cols = lax.broadcasted_iota(jnp.int32, (1, 8), 1)
    zv = jnp.where(cols < C, zv, 0.0)

    @pl.when(r == 0)
    def _():
        z_ref[0] = zv

    @pl.when(r > 0)
    def _():
        z_ref[0] = jnp.maximum(z_ref[0], zv)


def _pool(agg2p, degp, ha, hb, wl2a, wl2b, wr2a, wr2b, b2r):
    nb = NP // _BN
    return pl.pallas_call(
        _pool_body,
        grid=(T, nb),
        in_specs=[
            pl.BlockSpec((1, 1, 1, _BN, HD), lambda t, r: (0, t, 0, r, 0)),
            pl.BlockSpec((1, 1, 1, _BN, HD), lambda t, r: (0, t, 1, r, 0)),
            pl.BlockSpec((1, 1, 1, _BN, HD), lambda t, r: (1, t, 0, r, 0)),
            pl.BlockSpec((1, 1, 1, _BN, HD), lambda t, r: (1, t, 1, r, 0)),
            pl.BlockSpec((1, 1, _BN, 8), lambda t, r: (t, 0, r, 0)),
            pl.BlockSpec((1, 1, _BN, 8), lambda t, r: (t, 1, r, 0)),
            pl.BlockSpec((1, _BN, HD), lambda t, r: (t, r, 0)),
            pl.BlockSpec((1, _BN, HD), lambda t, r: (t, r, 0)),
            pl.BlockSpec((HD, 8), lambda t, r: (0, 0)),
            pl.BlockSpec((HD, 8), lambda t, r: (0, 0)),
            pl.BlockSpec((HD, 8), lambda t, r: (0, 0)),
            pl.BlockSpec((HD, 8), lambda t, r: (0, 0)),
            pl.BlockSpec((1, 8), lambda t, r: (0, 0)),
        ],
        out_specs=pl.BlockSpec((1, 1, 8), lambda t, r: (t, 0, 0)),
        out_shape=jax.ShapeDtypeStruct((T, 1, 8), jnp.float32),
    )(agg2p, agg2p, agg2p, agg2p, degp, degp, ha, hb, wl2a, wl2b,
      wr2a, wr2b, b2r)


def _head_body(z_ref, wih_ref, bih_ref, bhh_ref, ct_ref, rk_ref, q_ref,
               misc_ref, h1_ref):
    z = z_ref[...]
    gi = []
    for k in range(3):
        gi.append(jnp.dot(z, wih_ref[k], preferred_element_type=jnp.float32)
                  + bih_ref[k])
    r = jax.nn.sigmoid(gi[0] + bhh_ref[0])
    zt = jax.nn.sigmoid(gi[1] + bhh_ref[1])
    n = jnp.tanh(gi[2] + r * bhh_ref[2])
    h1 = (1.0 - zt) * n
    h1_ref[...] = h1
    s = lax.dot_general(z, h1, (((1,), (1,)), ((), ())),
                        preferred_element_type=jnp.float32)
    scol = jnp.dot(s, ct_ref[...], preferred_element_type=jnp.float32,
                   precision=_PH)
    q = q_ref[...]
    cols = []
    for k in range(len(_OFFS)):
        vk = jnp.dot(rk_ref[k], scol, preferred_element_type=jnp.float32,
                     precision=_PH)
        cols.append(jnp.sum(vk * q, axis=1, keepdims=True))
    v = jnp.concatenate(cols, axis=1)
    colv = lax.broadcasted_iota(jnp.int32, (_CNT, 8), 1)
    m = jnp.max(v, axis=1, keepdims=True)
    lse = jnp.log(jnp.sum(jnp.exp(v - m), axis=1, keepdims=True)) + m
    vpos = jnp.sum(jnp.where(colv == 0, v, 0.0), axis=1, keepdims=True)
    nce = jnp.sum(vpos - lse) / (-1.0 * _CNT)
    maxneg = jnp.max(jnp.where(colv >= 1, v, -jnp.inf), axis=1, keepdims=True)
    acc = jnp.sum((vpos >= maxneg).astype(jnp.float32)) / _CNT
    ri = lax.broadcasted_iota(jnp.int32, (8, 128), 0)
    ci = lax.broadcasted_iota(jnp.int32, (8, 128), 1)
    blk = jnp.where((ri == 0) & (ci == 0), nce, 0.0)
    blk = jnp.where((ri == 0) & (ci == 1), acc, blk)
    misc_ref[...] = blk


def _head(zp, wih3, bih3, bhh3, ct, rk, q):
    return pl.pallas_call(
        _head_body,
        out_shape=[
            jax.ShapeDtypeStruct((8, 128), jnp.float32),
            jax.ShapeDtypeStruct((_ZROWS, 8), jnp.float32),
        ],
    )(zp, wih3, bih3, bhh3, ct, rk, q)


def _head_consts():
    ct = np.zeros((_ZROWS, _NTS), np.float32)
    q = np.zeros((_CNT, _NTS), np.float32)
    rk = np.zeros((len(_OFFS), _CNT, _ZROWS), np.float32)
    for qi in range(_NTS):
        ts = _TS0 + qi
        ct[ts, qi] = 1.0
        for i in range(1, _NI + 1):
            p = qi * _NI + (i - 1)
            q[p, qi] = 1.0
            for k, off in enumerate(_OFFS):
                rk[k, p, ts + i + off] = 1.0
    return jnp.asarray(ct), jnp.asarray(rk), jnp.asarray(q)


@jax.jit
def kernel(x, edge_index, batch, Wl1, Wr1, b1, Wl2, Wr2, b2,
           W_ih, W_hh, b_ih, b_hh):
    f32 = jnp.float32
    xp = jnp.pad(x, ((0, 0), (0, NP - N), (0, 0)))
    xa = xp[:, :, :HD].reshape(T * NP, HD)
    xb = xp[:, :, HD:].reshape(T * NP, HD)
    src = edge_index[:, 0, :]
    dst = edge_index[:, 1, :]
    toff = (jnp.arange(T, dtype=jnp.int32) * NP)[:, None, None]
    src3 = src.reshape(T, NW, EW) + toff
    fsrc = jnp.broadcast_to(toff + N, (T, NW, EWP - EW))
    srco = jnp.concatenate([src3, fsrc], axis=2).reshape(T, NW, NCH, CH)
    dst3 = dst.reshape(T, NW, EW)
    fdst = jnp.full((T, NW, EWP - EW), N, jnp.int32)
    dstr = jnp.concatenate([dst3, fdst], axis=2).reshape(T, NW, NCH, CH)

    zrows = jnp.zeros((STRIPE, HD), f32)
    zrows8 = jnp.zeros((STRIPE, 8), f32)
    ones = jnp.ones((CH, 8), f32)

    aggp, degp = _segsum(xa, xb, srco, dstr, zrows, zrows8, ones, True)

    wl1t = Wl1.T
    wr1t = Wr1.T
    b1r = b1.reshape(1, D)

    ha, hb = _dense1(aggp, degp, xp, wl1t, wr1t, b1r)

    (agg2p,) = _segsum(ha.reshape(T * NP, HD), hb.reshape(T * NP, HD),
                       srco, dstr, zrows, zrows8, ones, False)

    wl2a = jnp.zeros((HD, 8), f32).at[:, :C].set(Wl2.T[:HD])
    wl2b = jnp.zeros((HD, 8), f32).at[:, :C].set(Wl2.T[HD:])
    wr2a = jnp.zeros((HD, 8), f32).at[:, :C].set(Wr2.T[:HD])
    wr2b = jnp.zeros((HD, 8), f32).at[:, :C].set(Wr2.T[HD:])
    b2r = jnp.zeros((1, 8), f32).at[0, :C].set(b2)

    z = _pool(agg2p, degp, ha, hb, wl2a, wl2b, wr2a, wr2b, b2r)
    zp = jnp.pad(z.reshape(T, 8), ((0, _ZROWS - T), (0, 0)))

    wih3 = jnp.zeros((3, 8, 8), f32)
    for k in range(3):
        wih3 = wih3.at[k, :C, :C].set(W_ih[C * k:C * (k + 1), :].T)
    bih3 = jnp.zeros((3, 1, 8), f32).at[:, 0, :C].set(b_ih.reshape(3, C))
    bhh3 = jnp.zeros((3, 1, 8), f32).at[:, 0, :C].set(b_hh.reshape(3, C))
    ct, rk, q = _head_consts()

    misc, h1 = _head(zp, wih3, bih3, bhh3, ct, rk, q)
    nce = misc[0, 0].reshape(1)
    acc = misc[0, 1]
    hidden = h1[:_TS0 + _NTS, :C][None]
    return nce, acc, hidden


# speedup vs baseline: 1.5404x; 1.5404x over previous
import jax
import jax.numpy as jnp
import numpy as np
from jax import lax
from jax.experimental import pallas as pl
from jax.experimental.pallas import tpu as pltpu
from jax.experimental.pallas import tpu_sc as plsc

T, N, E, D, C = 36, 10000, 320000, 128, 6
_PH = jax.lax.Precision.HIGHEST
NP = 10240
HD = D // 2
NC, NS = 2, 16
NW = NC * NS
EW = E // NW
CH = 80
NCH = 126
EWP = NCH * CH
STRIPE = NP // NS
W2 = 16

_TS0, _NTS, _NI = 6, 4, 12
_CNT = _NTS * _NI
_ZROWS = 40
_OFFS = (0, 7, 8, 9, 10, 11, 12, 13)


def _segsum(xa, xb, srco, dstr, zrows, zrows8, ones, with_deg):
    mesh = plsc.VectorSubcoreMesh(core_axis_name="c", subcore_axis_name="s")

    def body(xa_r, xb_r, srco_r, dstr_r, zrows_r, zrows8_r, ones_r,
             *out_and_scratch):
        if with_deg:
            (aggp_r, degp_r, idxs_v, idxd_v, rows_v, ones_v, zbuf_v, zbuf8_v,
             sem_a, sem_b, agg_sh, deg_sh) = out_and_scratch
        else:
            (aggp_r, idxs_v, idxd_v, rows_v, ones_v, zbuf_v, zbuf8_v,
             sem_a, sem_b, agg_sh) = out_and_scratch
            degp_r = deg_sh = None
        c = lax.axis_index("c")
        s = lax.axis_index("s")
        wid = s * NC + c
        pltpu.sync_copy(ones_r, ones_v)
        pltpu.sync_copy(zrows_r, zbuf_v)
        pltpu.sync_copy(zrows8_r, zbuf8_v)

        def step_t(t, carry):
            pltpu.sync_copy(srco_r.at[t, wid], idxs_v)
            pltpu.sync_copy(dstr_r.at[t, wid], idxd_v)
            for h, tab in ((0, xa_r), (1, xb_r)):
                pltpu.sync_copy(zbuf_v, agg_sh.at[pl.ds(s * STRIPE, STRIPE)])
                if h == 0 and with_deg:
                    pltpu.sync_copy(zbuf8_v,
                                    deg_sh.at[pl.ds(s * STRIPE, STRIPE)])
                plsc.subcore_barrier()

                pltpu.async_copy(tab.at[idxs_v.at[0]], rows_v.at[0], sem_a)

                def wait_rows(sem):
                    pltpu.make_async_copy(tab.at[idxs_v.at[0]],
                                          rows_v.at[0], sem).wait()

                def scat(j, buf):
                    pltpu.sync_copy(rows_v.at[buf], agg_sh.at[idxd_v.at[j]],
                                    add=True)
                    if h == 0 and with_deg:
                        pltpu.sync_copy(ones_v, deg_sh.at[idxd_v.at[j]],
                                        add=True)

                def step_j(jj, carry2):
                    j0 = 2 * jj
                    pltpu.async_copy(tab.at[idxs_v.at[j0 + 1]],
                                     rows_v.at[1], sem_b)
                    wait_rows(sem_a)
                    scat(j0, 0)

                    @pl.when(jj < NCH // 2 - 1)
                    def _():
                        pltpu.async_copy(tab.at[idxs_v.at[j0 + 2]],
                                         rows_v.at[0], sem_a)

                    wait_rows(sem_b)
                    scat(j0 + 1, 1)
                    return carry2

                lax.fori_loop(0, NCH // 2, step_j, 0)
                plsc.subcore_barrier()
                pltpu.sync_copy(agg_sh.at[pl.ds(s * STRIPE, STRIPE)],
                                aggp_r.at[h, t, c, pl.ds(s * STRIPE, STRIPE)])
                if h == 0 and with_deg:
                    pltpu.sync_copy(deg_sh.at[pl.ds(s * STRIPE, STRIPE)],
                                    degp_r.at[t, c, pl.ds(s * STRIPE, STRIPE)])
                plsc.subcore_barrier()
            return carry

        lax.fori_loop(0, T, step_t, 0)

    out_type = [jax.ShapeDtypeStruct((2, T, NC, NP, HD), jnp.float32)]
    scratch = [
        pltpu.VMEM((NCH, CH), jnp.int32),
        pltpu.VMEM((NCH, CH), jnp.int32),
        pltpu.VMEM((2, CH, HD), jnp.float32),
        pltpu.VMEM((CH, 8), jnp.float32),
        pltpu.VMEM((STRIPE, HD), jnp.float32),
        pltpu.VMEM((STRIPE, 8), jnp.float32),
        pltpu.SemaphoreType.DMA,
        pltpu.SemaphoreType.DMA,
        pltpu.VMEM_SHARED((NP, HD), jnp.float32),
    ]
    if with_deg:
        out_type.append(jax.ShapeDtypeStruct((T, NC, NP, 8), jnp.float32))
        scratch.append(pltpu.VMEM_SHARED((NP, 8), jnp.float32))

    f = pl.kernel(
        body,
        out_type=tuple(out_type),
        mesh=mesh,
        scratch_types=scratch,
        compiler_params=pltpu.CompilerParams(use_tc_tiling_on_sc=False),
    )
    return f(xa, xb, srco, dstr, zrows, zrows8, ones)


_BN = 2048


def _dense1_body(a00, a01, a10, a11, d0, d1, x, wl1t, wr1t, b1,
                 ha_ref, hb_ref):
    agg = jnp.concatenate([a00[0, 0, 0] + a01[0, 0, 0],
                           a10[0, 0, 0] + a11[0, 0, 0]], axis=1)
    deg = d0[0, 0, :, 0:1] + d1[0, 0, :, 0:1]
    aggm = agg / jnp.maximum(deg, 1.0)
    h = (jnp.dot(aggm, wl1t[...], preferred_element_type=jnp.float32)
         + jnp.dot(x[0], wr1t[...], preferred_element_type=jnp.float32)
         + b1[...])
    h = jnp.maximum(h, 0.0)
    ha_ref[0] = h[:, :HD]
    hb_ref[0] = h[:, HD:]


def _dense1(aggp, degp, xp, wl1t, wr1t, b1r):
    nb = NP // _BN
    return pl.pallas_call(
        _dense1_body,
        grid=(T, nb),
        in_specs=[
            pl.BlockSpec((1, 1, 1, _BN, HD), lambda t, r: (0, t, 0, r, 0)),
            pl.BlockSpec((1, 1, 1, _BN, HD), lambda t, r: (0, t, 1, r, 0)),
            pl.BlockSpec((1, 1, 1, _BN, HD), lambda t, r: (1, t, 0, r, 0)),
            pl.BlockSpec((1, 1, 1, _BN, HD), lambda t, r: (1, t, 1, r, 0)),
            pl.BlockSpec((1, 1, _BN, 8), lambda t, r: (t, 0, r, 0)),
            pl.BlockSpec((1, 1, _BN, 8), lambda t, r: (t, 1, r, 0)),
            pl.BlockSpec((1, _BN, D), lambda t, r: (t, r, 0)),
            pl.BlockSpec((D, D), lambda t, r: (0, 0)),
            pl.BlockSpec((D, D), lambda t, r: (0, 0)),
            pl.BlockSpec((1, D), lambda t, r: (0, 0)),
        ],
        out_specs=[
            pl.BlockSpec((1, _BN, HD), lambda t, r: (t, r, 0)),
            pl.BlockSpec((1, _BN, HD), lambda t, r: (t, r, 0)),
        ],
        out_shape=[
            jax.ShapeDtypeStruct((T, NP, HD), jnp.float32),
            jax.ShapeDtypeStruct((T, NP, HD), jnp.float32),
        ],
    )(aggp, aggp, aggp, aggp, degp, degp, xp, wl1t, wr1t, b1r)


def _pool_body(a00, a01, a10, a11, d0, d1, ha, hb, wl2a, wl2b, wr2a, wr2b,
               b2, z_ref):
    r = pl.program_id(1)
    deg = jnp.maximum(d0[0, 0, :, 0:1] + d1[0, 0, :, 0:1], 1.0)
    agg_a = (a00[0, 0, 0] + a01[0, 0, 0]) / deg
    agg_b = (a10[0, 0, 0] + a11[0, 0, 0]) / deg
    out2 = (jnp.dot(agg_a, wl2a[...], preferred_element_type=jnp.float32)
            + jnp.dot(agg_b, wl2b[...], preferred_element_type=jnp.float32)
            + jnp.dot(ha[0], wr2a[...], preferred_element_type=jnp.float32)
            + jnp.dot(hb[0], wr2b[...], preferred_element_type=jnp.float32)
            + b2[...])
    rows = lax.broadcasted_iota(jnp.int32, (_BN, 8), 0) + r * _BN
    out2 = jnp.where(rows < N, out2, -jnp.inf)
    zv = jnp.max(out2, axis=0, keepdims=True)
    cols = lax.broadcasted_iota(jnp.int32, (1, 8), 1)
    zv = jnp.where(cols < C, zv, 0.0)

    @pl.when(r == 0)
    def _():
        z_ref[0] = zv

    @pl.when(r > 0)
    def _():
        z_ref[0] = jnp.maximum(z_ref[0], zv)


def _pool(agg2p, degp, ha, hb, wl2a, wl2b, wr2a, wr2b, b2r):
    nb = NP // _BN
    return pl.pallas_call(
        _pool_body,
        grid=(T, nb),
        in_specs=[
            pl.BlockSpec((1, 1, 1, _BN, HD), lambda t, r: (0, t, 0, r, 0)),
            pl.BlockSpec((1, 1, 1, _BN, HD), lambda t, r: (0, t, 1, r, 0)),
            pl.BlockSpec((1, 1, 1, _BN, HD), lambda t, r: (1, t, 0, r, 0)),
            pl.BlockSpec((1, 1, 1, _BN, HD), lambda t, r: (1, t, 1, r, 0)),
            pl.BlockSpec((1, 1, _BN, 8), lambda t, r: (t, 0, r, 0)),
            pl.BlockSpec((1, 1, _BN, 8), lambda t, r: (t, 1, r, 0)),
            pl.BlockSpec((1, _BN, HD), lambda t, r: (t, r, 0)),
            pl.BlockSpec((1, _BN, HD), lambda t, r: (t, r, 0)),
            pl.BlockSpec((HD, 8), lambda t, r: (0, 0)),
            pl.BlockSpec((HD, 8), lambda t, r: (0, 0)),
            pl.BlockSpec((HD, 8), lambda t, r: (0, 0)),
            pl.BlockSpec((HD, 8), lambda t, r: (0, 0)),
            pl.BlockSpec((1, 8), lambda t, r: (0, 0)),
        ],
        out_specs=pl.BlockSpec((1, 1, 8), lambda t, r: (t, 0, 0)),
        out_shape=jax.ShapeDtypeStruct((T, 1, 8), jnp.float32),
    )(agg2p, agg2p, agg2p, agg2p, degp, degp, ha, hb, wl2a, wl2b,
      wr2a, wr2b, b2r)


def _head_body(z_ref, wih_ref, bih_ref, bhh_ref, ct_ref, rk_ref, q_ref,
               misc_ref, h1_ref):
    z = z_ref[...]
    gi = []
    for k in range(3):
        gi.append(jnp.dot(z, wih_ref[k], preferred_element_type=jnp.float32)
                  + bih_ref[k])
    r = jax.nn.sigmoid(gi[0] + bhh_ref[0])
    zt = jax.nn.sigmoid(gi[1] + bhh_ref[1])
    n = jnp.tanh(gi[2] + r * bhh_ref[2])
    h1 = (1.0 - zt) * n
    h1_ref[...] = h1
    s = lax.dot_general(z, h1, (((1,), (1,)), ((), ())),
                        preferred_element_type=jnp.float32)
    scol = jnp.dot(s, ct_ref[...], preferred_element_type=jnp.float32,
                   precision=_PH)
    q = q_ref[...]
    cols = []
    for k in range(len(_OFFS)):
        vk = jnp.dot(rk_ref[k], scol, preferred_element_type=jnp.float32,
                     precision=_PH)
        cols.append(jnp.sum(vk * q, axis=1, keepdims=True))
    v = jnp.concatenate(cols, axis=1)
    colv = lax.broadcasted_iota(jnp.int32, (_CNT, 8), 1)
    m = jnp.max(v, axis=1, keepdims=True)
    lse = jnp.log(jnp.sum(jnp.exp(v - m), axis=1, keepdims=True)) + m
    vpos = jnp.sum(jnp.where(colv == 0, v, 0.0), axis=1, keepdims=True)
    nce = jnp.sum(vpos - lse) / (-1.0 * _CNT)
    maxneg = jnp.max(jnp.where(colv >= 1, v, -jnp.inf), axis=1, keepdims=True)
    acc = jnp.sum((vpos >= maxneg).astype(jnp.float32)) / _CNT
    ri = lax.broadcasted_iota(jnp.int32, (8, 128), 0)
    ci = lax.broadcasted_iota(jnp.int32, (8, 128), 1)
    blk = jnp.where((ri == 0) & (ci == 0), nce, 0.0)
    blk = jnp.where((ri == 0) & (ci == 1), acc, blk)
    misc_ref[...] = blk


def _head(zp, wih3, bih3, bhh3, ct, rk, q):
    return pl.pallas_call(
        _head_body,
        out_shape=[
            jax.ShapeDtypeStruct((8, 128), jnp.float32),
            jax.ShapeDtypeStruct((_ZROWS, 8), jnp.float32),
        ],
    )(zp, wih3, bih3, bhh3, ct, rk, q)


def _head_consts():
    ct = np.zeros((_ZROWS, _NTS), np.float32)
    q = np.zeros((_CNT, _NTS), np.float32)
    rk = np.zeros((len(_OFFS), _CNT, _ZROWS), np.float32)
    for qi in range(_NTS):
        ts = _TS0 + qi
        ct[ts, qi] = 1.0
        for i in range(1, _NI + 1):
            p = qi * _NI + (i - 1)
            q[p, qi] = 1.0
            for k, off in enumerate(_OFFS):
                rk[k, p, ts + i + off] = 1.0
    return jnp.asarray(ct), jnp.asarray(rk), jnp.asarray(q)


@jax.jit
def kernel(x, edge_index, batch, Wl1, Wr1, b1, Wl2, Wr2, b2,
           W_ih, W_hh, b_ih, b_hh):
    f32 = jnp.float32
    xp = jnp.pad(x, ((0, 0), (0, NP - N), (0, 0)))
    xa = xp[:, :, :HD].reshape(T * NP, HD)
    xb = xp[:, :, HD:].reshape(T * NP, HD)
    src = edge_index[:, 0, :]
    dst = edge_index[:, 1, :]
    toff = (jnp.arange(T, dtype=jnp.int32) * NP)[:, None, None]
    src3 = src.reshape(T, NW, EW) + toff
    fsrc = jnp.broadcast_to(toff + N, (T, NW, EWP - EW))
    srco = jnp.concatenate([src3, fsrc], axis=2).reshape(T, NW, NCH, CH)
    dst3 = dst.reshape(T, NW, EW)
    fdst = jnp.full((T, NW, EWP - EW), N, jnp.int32)
    dstr = jnp.concatenate([dst3, fdst], axis=2).reshape(T, NW, NCH, CH)

    zrows = jnp.zeros((STRIPE, HD), f32)
    zrows8 = jnp.zeros((STRIPE, 8), f32)
    ones = jnp.ones((CH, 8), f32)

    aggp, degp = _segsum(xa, xb, srco, dstr, zrows, zrows8, ones, True)

    wl1t = Wl1.T
    wr1t = Wr1.T
    b1r = b1.reshape(1, D)

    ha, hb = _dense1(aggp, degp, xp, wl1t, wr1t, b1r)

    (agg2p,) = _segsum(ha.reshape(T * NP, HD), hb.reshape(T * NP, HD),
                       srco, dstr, zrows, zrows8, ones, False)

    wl2a = jnp.zeros((HD, 8), f32).at[:, :C].set(Wl2.T[:HD])
    wl2b = jnp.zeros((HD, 8), f32).at[:, :C].set(Wl2.T[HD:])
    wr2a = jnp.zeros((HD, 8), f32).at[:, :C].set(Wr2.T[:HD])
    wr2b = jnp.zeros((HD, 8), f32).at[:, :C].set(Wr2.T[HD:])
    b2r = jnp.zeros((1, 8), f32).at[0, :C].set(b2)

    z = _pool(agg2p, degp, ha, hb, wl2a, wl2b, wr2a, wr2b, b2r)
    zp = jnp.pad(z.reshape(T, 8), ((0, _ZROWS - T), (0, 0)))

    wih3 = jnp.zeros((3, 8, 8), f32)
    for k in range(3):
        wih3 = wih3.at[k, :C, :C].set(W_ih[C * k:C * (k + 1), :].T)
    bih3 = jnp.zeros((3, 1, 8), f32).at[:, 0, :C].set(b_ih.reshape(3, C))
    bhh3 = jnp.zeros((3, 1, 8), f32).at[:, 0, :C].set(b_hh.reshape(3, C))
    ct, rk, q = _head_consts()

    misc, h1 = _head(zp, wih3, bih3, bhh3, ct, rk, q)
    nce = misc[0, 0].reshape(1)
    acc = misc[0, 1]
    hidden = h1[:_TS0 + _NTS, :C][None]
    return nce, acc, hidden
